# baseline (speedup 1.0000x reference)
import functools

import jax
import jax.numpy as jnp
from jax import lax
from jax.experimental import pallas as pl
from jax.experimental.pallas import tpu as pltpu
from jax.experimental.pallas import tpu_sc as plsc

N = 32768
E = 262144
LAT = 128
L = 16
G = LAT // L
NC, NS = 2, 16
NW = NC * NS
EPT = E // NW
CH = 256
NCHUNK = EPT // CH
JR = CH // 128
NODE_SL = N // NS
ZROWS = 512

_f32 = jnp.float32


def _node_proj_body(x_ref, wn_ref, bn_ref, ws_ref, wr_ref, h_ref, p_ref, q_ref):
    h = jnp.dot(x_ref[...], wn_ref[...], preferred_element_type=_f32) + bn_ref[...]
    h_ref[...] = h
    p_ref[...] = jnp.dot(h, ws_ref[...], preferred_element_type=_f32)
    q_ref[...] = jnp.dot(h, wr_ref[...], preferred_element_type=_f32)


def _node_proj(nodes, W_embed_n, b_embed_n, We1_s, We1_r):
    blk = 2048
    grid = (N // blk,)
    return pl.pallas_call(
        _node_proj_body,
        grid=grid,
        in_specs=[
            pl.BlockSpec((blk, LAT), lambda i: (i, 0)),
            pl.BlockSpec((LAT, LAT), lambda i: (0, 0)),
            pl.BlockSpec((1, LAT), lambda i: (0, 0)),
            pl.BlockSpec((LAT, LAT), lambda i: (0, 0)),
            pl.BlockSpec((LAT, LAT), lambda i: (0, 0)),
        ],
        out_specs=[
            pl.BlockSpec((blk, LAT), lambda i: (i, 0)),
            pl.BlockSpec((blk, LAT), lambda i: (i, 0)),
            pl.BlockSpec((blk, LAT), lambda i: (i, 0)),
        ],
        out_shape=[jax.ShapeDtypeStruct((N, LAT), _f32)] * 3,
    )(nodes, W_embed_n, b_embed_n, We1_s, We1_r)


def _edge_proj_body(x_ref, we_ref, be_ref, w1_ref, b1_ref, out_ref):
    h = jnp.dot(x_ref[...], we_ref[...], preferred_element_type=_f32) + be_ref[...]
    ep = jnp.dot(h, w1_ref[...], preferred_element_type=_f32) + b1_ref[...]
    ep3 = ep.reshape(ep.shape[0] // 2, 2, LAT)
    ai = jax.lax.bitcast_convert_type(ep3[:, 0, :], jnp.int32)
    bi = jax.lax.bitcast_convert_type(ep3[:, 1, :], jnp.int32)
    out_ref[...] = (ai & jnp.int32(-65536)) | jax.lax.shift_right_logical(bi, 16)


def _edge_proj(edges, W_embed_e, b_embed_e, We1_e, be1):
    blk = 4096
    grid = (E // blk,)
    return pl.pallas_call(
        _edge_proj_body,
        grid=grid,
        in_specs=[
            pl.BlockSpec((blk, 16), lambda i: (i, 0)),
            pl.BlockSpec((16, LAT), lambda i: (0, 0)),
            pl.BlockSpec((1, LAT), lambda i: (0, 0)),
            pl.BlockSpec((LAT, LAT), lambda i: (0, 0)),
            pl.BlockSpec((1, LAT), lambda i: (0, 0)),
        ],
        out_specs=pl.BlockSpec((blk // 2, LAT), lambda i: (i, 0)),
        out_shape=jax.ShapeDtypeStruct((E // 2, LAT), jnp.int32),
    )(edges, W_embed_e, b_embed_e, We1_e, be1)


def _node_mlp_body(h_ref, s_ref, r_ref, d_ref, wn1n_ref, we2_ref, wn1s_ref,
                   wn1r_ref, be2_ref, bn1_ref, wn2_ref, bn2_ref, out_ref):
    S = s_ref[0] + s_ref[1]
    R = r_ref[0] + r_ref[1]
    ds = d_ref[0, 0, :, 0:1] + d_ref[1, 0, :, 0:1]
    dr = d_ref[0, 1, :, 0:1] + d_ref[1, 1, :, 0:1]
    Ms = jnp.dot(we2_ref[...], wn1s_ref[...], preferred_element_type=_f32)
    Mr = jnp.dot(we2_ref[...], wn1r_ref[...], preferred_element_type=_f32)
    bs = jnp.dot(be2_ref[...], wn1s_ref[...], preferred_element_type=_f32)
    br = jnp.dot(be2_ref[...], wn1r_ref[...], preferred_element_type=_f32)
    pre = (jnp.dot(h_ref[...], wn1n_ref[...], preferred_element_type=_f32)
           + jnp.dot(S, Ms, preferred_element_type=_f32)
           + jnp.dot(R, Mr, preferred_element_type=_f32)
           + ds * bs + dr * br + bn1_ref[...])
    nh = jnp.maximum(pre, 0.0)
    out_ref[...] = jnp.dot(nh, wn2_ref[...], preferred_element_type=_f32) + bn2_ref[...]


def _node_mlp(h_nodes, S2, R2, D2, Wn1_n, We2, Wn1_s, Wn1_r, be2, bn1, Wn2, bn2):
    blk = 2048
    grid = (N // blk,)
    wspec = pl.BlockSpec((LAT, LAT), lambda i: (0, 0))
    bspec = pl.BlockSpec((1, LAT), lambda i: (0, 0))
    return pl.pallas_call(
        _node_mlp_body,
        grid=grid,
        in_specs=[
            pl.BlockSpec((blk, LAT), lambda i: (i, 0)),
            pl.BlockSpec((NC, blk, LAT), lambda i: (0, i, 0)),
            pl.BlockSpec((NC, blk, LAT), lambda i: (0, i, 0)),
            pl.BlockSpec((NC, 2, blk, L), lambda i: (0, 0, i, 0)),
            wspec, wspec, wspec, wspec, bspec, bspec, wspec, bspec,
        ],
        out_specs=pl.BlockSpec((blk, LAT), lambda i: (i, 0)),
        out_shape=jax.ShapeDtypeStruct((N, LAT), _f32),
    )(h_nodes, S2, R2, D2, Wn1_n, We2, Wn1_s, Wn1_r, be2, bn1, Wn2, bn2)


def _enc1_body(x_ref, w_ref, b_ref, out_ref):
    k = pl.program_id(0)

    @pl.when(k == 0)
    def _():
        out_ref[...] = jnp.zeros_like(out_ref)

    out_ref[...] += jnp.dot(x_ref[...], w_ref[...], preferred_element_type=_f32)

    @pl.when(k == pl.num_programs(0) - 1)
    def _():
        out_ref[...] = jnp.maximum(out_ref[...] + b_ref[...], 0.0)


def _enc1(X, Wenc1, benc1):
    kblk = 4096
    grid = (X.shape[1] // kblk,)
    return pl.pallas_call(
        _enc1_body,
        grid=grid,
        in_specs=[
            pl.BlockSpec((64, kblk), lambda k: (0, k)),
            pl.BlockSpec((kblk, 256), lambda k: (k, 0)),
            pl.BlockSpec((1, 256), lambda k: (0, 0)),
        ],
        out_specs=pl.BlockSpec((64, 256), lambda k: (0, 0)),
        out_shape=jax.ShapeDtypeStruct((64, 256), _f32),
    )(X, Wenc1, benc1)


def _enc2_body(x_ref, w_ref, b_ref, out_ref):
    out_ref[...] = jnp.maximum(
        jnp.dot(x_ref[...], w_ref[...], preferred_element_type=_f32) + b_ref[...], 0.0)


def _enc2(X, Wenc2, benc2):
    return pl.pallas_call(
        _enc2_body,
        out_shape=jax.ShapeDtypeStruct((64, 128), _f32),
    )(X, Wenc2, benc2)


def _sc_body(sidx_hbm, ridx_hbm, zeros_hbm, ep_hbm, p_hbm, q_hbm,
             s_out, r_out, d_out,
             s_idx, r_idx, fsb, frb, pb, qb, epb, ehb, shS, shR,
             sem_g0, sem_g1, sem_s0, sem_s1, sem_d):
    cid = lax.axis_index("c")
    sid = lax.axis_index("s")
    wid = sid * NC + cid
    row0 = wid * (EPT // 128)
    sem_g = (sem_g0, sem_g1)
    sem_s = (sem_s0, sem_s1)

    pltpu.sync_copy(sidx_hbm.at[pl.ds(wid * EPT, EPT)], s_idx)
    pltpu.sync_copy(ridx_hbm.at[pl.ds(wid * EPT, EPT)], r_idx)

    def _clear_tables():
        pltpu.async_copy(zeros_hbm, shS.at[pl.ds(sid * NODE_SL, NODE_SL)], sem_d)
        pltpu.async_copy(zeros_hbm, shR.at[pl.ds(sid * NODE_SL, NODE_SL)], sem_d)
        pltpu.make_async_copy(zeros_hbm, shS.at[pl.ds(sid * NODE_SL, NODE_SL)],
                              sem_d).wait()
        pltpu.make_async_copy(zeros_hbm, shR.at[pl.ds(sid * NODE_SL, NODE_SL)],
                              sem_d).wait()

    _clear_tables()
    plsc.subcore_barrier()

    def _drain_group(g):
        nrow = sid * NODE_SL
        pltpu.async_copy(shS.at[pl.ds(nrow, NODE_SL)],
                         s_out.at[cid, pl.ds(nrow, NODE_SL), pl.ds(g * L, L)],
                         sem_d)
        pltpu.async_copy(shR.at[pl.ds(nrow, NODE_SL)],
                         r_out.at[cid, pl.ds(nrow, NODE_SL), pl.ds(g * L, L)],
                         sem_d)
        pltpu.make_async_copy(
            shS.at[pl.ds(nrow, NODE_SL)],
            s_out.at[cid, pl.ds(nrow, NODE_SL), pl.ds(g * L, L)], sem_d).wait()
        pltpu.make_async_copy(
            shR.at[pl.ds(nrow, NODE_SL)],
            r_out.at[cid, pl.ds(nrow, NODE_SL), pl.ds(g * L, L)], sem_d).wait()
        _clear_tables()
        plsc.subcore_barrier()

    for g in range(G):
        def _ep_src(ch):
            p0 = (wid * EPT + ch * CH) // 2
            return ep_hbm.at[pl.ds(p0, CH // 2), pl.ds(g * L, L)]

        def _issue(ch, par):
            def _fidx(c, _):
                sl = pl.ds(c * 16, 16)
                src_sl = pl.ds(ch * CH + c * 16, 16)
                fsb[par, sl] = s_idx[src_sl] * G + g
                frb[par, sl] = r_idx[src_sl] * G + g
                return _
            lax.fori_loop(0, CH // 16, _fidx, None)
            pltpu.async_copy(_ep_src(ch), epb.at[par], sem_g[par])
            pltpu.async_copy(p_hbm.at[fsb.at[par]], pb.at[par], sem_g[par])
            pltpu.async_copy(q_hbm.at[frb.at[par]], qb.at[par], sem_g[par])

        def _wait_gathers(ch, par):
            pltpu.make_async_copy(_ep_src(ch), epb.at[par], sem_g[par]).wait()
            pltpu.make_async_copy(p_hbm.at[fsb.at[par]], pb.at[par],
                                  sem_g[par]).wait()
            pltpu.make_async_copy(q_hbm.at[frb.at[par]], qb.at[par],
                                  sem_g[par]).wait()

        def _wait_scatters(ch, par):
            pltpu.make_async_copy(ehb.at[par],
                                  shS.at[s_idx.at[pl.ds(ch * CH, CH)]],
                                  sem_s[par]).wait()
            pltpu.make_async_copy(ehb.at[par],
                                  shR.at[r_idx.at[pl.ds(ch * CH, CH)]],
                                  sem_s[par]).wait()

        def _process(ch, par):
            _wait_gathers(ch, par)

            @pl.when(ch >= 2)
            def _():
                _wait_scatters(ch - 2, par)

            def _rows(i, _):
                for u in range(2):
                    rp = i * 2 + u
                    v = epb[par, rp]
                    e0 = plsc.bitcast(v & jnp.int32(-65536), _f32)
                    e1 = plsc.bitcast(jax.lax.shift_left(v, 16), _f32)
                    ehb[par, 2 * rp] = jnp.maximum(
                        pb[par, 2 * rp] + qb[par, 2 * rp] + e0, 0.0)
                    ehb[par, 2 * rp + 1] = jnp.maximum(
                        pb[par, 2 * rp + 1] + qb[par, 2 * rp + 1] + e1, 0.0)
                return _
            lax.fori_loop(0, CH // 4, _rows, None)

            pltpu.async_copy(ehb.at[par], shS.at[s_idx.at[pl.ds(ch * CH, CH)]],
                             sem_s[par], add=True)
            pltpu.async_copy(ehb.at[par], shR.at[r_idx.at[pl.ds(ch * CH, CH)]],
                             sem_s[par], add=True)

        _issue(0, 0)
        if g > 0:
            _drain_group(g - 1)

        def _pipe(i, _):
            c0 = i * 2
            _issue(c0 + 1, 1)
            _process(c0, 0)

            @pl.when(c0 + 2 < NCHUNK)
            def _():
                _issue(c0 + 2, 0)
            _process(c0 + 1, 1)
            return _
        lax.fori_loop(0, NCHUNK // 2, _pipe, None)
        _wait_scatters(NCHUNK - 2, 0)
        _wait_scatters(NCHUNK - 1, 1)
        plsc.subcore_barrier()

    _drain_group(G - 1)

    one0 = jnp.where(lax.iota(jnp.int32, L) == 0, 1.0, 0.0).astype(_f32)

    def _preset(i, _):
        pb[0, i] = one0
        return _
    lax.fori_loop(0, CH, _preset, None)

    def _dchunk(ch, _):
        pltpu.async_copy(pb.at[0], shS.at[s_idx.at[pl.ds(ch * CH, CH)]],
                         sem_s0, add=True)
        pltpu.async_copy(pb.at[0], shR.at[r_idx.at[pl.ds(ch * CH, CH)]],
                         sem_s0, add=True)
        return _
    lax.fori_loop(0, NCHUNK, _dchunk, None)

    def _ddrain(ch, _):
        pltpu.make_async_copy(pb.at[0], shS.at[s_idx.at[pl.ds(ch * CH, CH)]],
                              sem_s0).wait()
        pltpu.make_async_copy(pb.at[0], shR.at[r_idx.at[pl.ds(ch * CH, CH)]],
                              sem_s0).wait()
        return _
    lax.fori_loop(0, NCHUNK, _ddrain, None)
    plsc.subcore_barrier()
    nrow = sid * NODE_SL
    pltpu.sync_copy(shS.at[pl.ds(nrow, NODE_SL)], d_out.at[cid, 0, pl.ds(nrow, NODE_SL)])
    pltpu.sync_copy(shR.at[pl.ds(nrow, NODE_SL)], d_out.at[cid, 1, pl.ds(nrow, NODE_SL)])


@functools.cache
def _make_sc_kernel():
    return functools.partial(
        pl.kernel,
        out_type=(
            jax.ShapeDtypeStruct((NC, N, LAT), _f32),
            jax.ShapeDtypeStruct((NC, N, LAT), _f32),
            jax.ShapeDtypeStruct((NC, 2, N, L), _f32),
        ),
        mesh=plsc.VectorSubcoreMesh(core_axis_name="c", subcore_axis_name="s"),
        scratch_types=(
            pltpu.VMEM((EPT,), jnp.int32),
            pltpu.VMEM((EPT,), jnp.int32),
            pltpu.VMEM((2, CH), jnp.int32),
            pltpu.VMEM((2, CH), jnp.int32),
            pltpu.VMEM((2, CH, L), _f32),
            pltpu.VMEM((2, CH, L), _f32),
            pltpu.VMEM((2, CH // 2, L), jnp.int32),
            pltpu.VMEM((2, CH, L), _f32),
            pltpu.VMEM_SHARED((N, L), _f32),
            pltpu.VMEM_SHARED((N, L), _f32),
            pltpu.SemaphoreType.DMA,
            pltpu.SemaphoreType.DMA,
            pltpu.SemaphoreType.DMA,
            pltpu.SemaphoreType.DMA,
            pltpu.SemaphoreType.DMA,
        ),
        compiler_params=pltpu.CompilerParams(use_tc_tiling_on_sc=False, needs_layout_passes=False),
    )(_sc_body)


def _sc_edge_pass(sidx, ridx, zeros_in, ep, p_flat, q_flat):
    return _make_sc_kernel()(sidx, ridx, zeros_in, ep, p_flat, q_flat)


def kernel(nodes, edges, senders, receivers, n_node, n_edge,
           W_embed_n, b_embed_n, W_embed_e, b_embed_e,
           We1, be1, We2, be2, Wn1, bn1, Wn2, bn2,
           Wenc1, benc1, Wenc2, benc2):
    We1_e, We1_s, We1_r = We1[0:128], We1[128:256], We1[256:384]
    Wn1_n, Wn1_s, Wn1_r = Wn1[0:128], Wn1[128:256], Wn1[256:384]

    h_nodes, P, Q = _node_proj(nodes, W_embed_n, b_embed_n.reshape(1, LAT),
                               We1_s, We1_r)
    Ep = _edge_proj(edges, W_embed_e, b_embed_e.reshape(1, LAT), We1_e,
                    be1.reshape(1, LAT))

    zeros_in = jnp.zeros((NODE_SL, L), _f32)

    S2, R2, D2 = _sc_edge_pass(senders, receivers, zeros_in, Ep,
                               P.reshape(N * G, L), Q.reshape(N * G, L))

    new_nodes = _node_mlp(h_nodes, S2, R2, D2, Wn1_n, We2, Wn1_s, Wn1_r,
                          be2.reshape(1, LAT), bn1.reshape(1, LAT), Wn2,
                          bn2.reshape(1, LAT))

    X = _enc1(new_nodes.reshape(64, 512 * LAT), Wenc1, benc1.reshape(1, 256))
    Y = _enc2(X, Wenc2, benc2.reshape(1, 128))
    return Y.reshape(8, 8, 128)


# speedup vs baseline: 1.0958x; 1.0958x over previous
import functools

import jax
import jax.numpy as jnp
from jax import lax
from jax.experimental import pallas as pl
from jax.experimental.pallas import tpu as pltpu
from jax.experimental.pallas import tpu_sc as plsc

N = 32768
E = 262144
LAT = 128
L = 16
G = LAT // L
NC, NS = 2, 16
NW = NC * NS
EPT = E // NW
CH = 256
NCHUNK = EPT // CH
JR = CH // 128
NODE_SL = N // NS
ZROWS = 512

_f32 = jnp.float32


def _node_proj_body(x_ref, wn_ref, bn_ref, ws_ref, wr_ref, h_ref, p_ref, q_ref):
    h = jnp.dot(x_ref[...], wn_ref[...], preferred_element_type=_f32) + bn_ref[...]
    h_ref[...] = h
    p_ref[...] = jnp.dot(h, ws_ref[...], preferred_element_type=_f32)
    q_ref[...] = jnp.dot(h, wr_ref[...], preferred_element_type=_f32)


def _node_proj(nodes, W_embed_n, b_embed_n, We1_s, We1_r):
    blk = 2048
    grid = (N // blk,)
    return pl.pallas_call(
        _node_proj_body,
        grid=grid,
        in_specs=[
            pl.BlockSpec((blk, LAT), lambda i: (i, 0)),
            pl.BlockSpec((LAT, LAT), lambda i: (0, 0)),
            pl.BlockSpec((1, LAT), lambda i: (0, 0)),
            pl.BlockSpec((LAT, LAT), lambda i: (0, 0)),
            pl.BlockSpec((LAT, LAT), lambda i: (0, 0)),
        ],
        out_specs=[
            pl.BlockSpec((blk, LAT), lambda i: (i, 0)),
            pl.BlockSpec((blk, LAT), lambda i: (i, 0)),
            pl.BlockSpec((blk, LAT), lambda i: (i, 0)),
        ],
        out_shape=[jax.ShapeDtypeStruct((N, LAT), _f32)] * 3,
    )(nodes, W_embed_n, b_embed_n, We1_s, We1_r)


def _edge_proj_body(x_ref, we_ref, be_ref, w1_ref, b1_ref, out_ref):
    h = jnp.dot(x_ref[...], we_ref[...], preferred_element_type=_f32) + be_ref[...]
    out_ref[...] = jnp.dot(h, w1_ref[...], preferred_element_type=_f32) + b1_ref[...]


def _edge_proj(edges, W_embed_e, b_embed_e, We1_e, be1):
    blk = 4096
    grid = (E // blk,)
    return pl.pallas_call(
        _edge_proj_body,
        grid=grid,
        in_specs=[
            pl.BlockSpec((blk, 16), lambda i: (i, 0)),
            pl.BlockSpec((16, LAT), lambda i: (0, 0)),
            pl.BlockSpec((1, LAT), lambda i: (0, 0)),
            pl.BlockSpec((LAT, LAT), lambda i: (0, 0)),
            pl.BlockSpec((1, LAT), lambda i: (0, 0)),
        ],
        out_specs=pl.BlockSpec((blk, LAT), lambda i: (i, 0)),
        out_shape=jax.ShapeDtypeStruct((E, LAT), _f32),
    )(edges, W_embed_e, b_embed_e, We1_e, be1)


def _node_mlp_body(h_ref, s_ref, r_ref, d_ref, wn1n_ref, we2_ref, wn1s_ref,
                   wn1r_ref, be2_ref, bn1_ref, wn2_ref, bn2_ref, out_ref):
    S = s_ref[0] + s_ref[1]
    R = r_ref[0] + r_ref[1]
    ds = d_ref[0, 0, :, 0:1] + d_ref[1, 0, :, 0:1]
    dr = d_ref[0, 1, :, 0:1] + d_ref[1, 1, :, 0:1]
    Ms = jnp.dot(we2_ref[...], wn1s_ref[...], preferred_element_type=_f32)
    Mr = jnp.dot(we2_ref[...], wn1r_ref[...], preferred_element_type=_f32)
    bs = jnp.dot(be2_ref[...], wn1s_ref[...], preferred_element_type=_f32)
    br = jnp.dot(be2_ref[...], wn1r_ref[...], preferred_element_type=_f32)
    pre = (jnp.dot(h_ref[...], wn1n_ref[...], preferred_element_type=_f32)
           + jnp.dot(S, Ms, preferred_element_type=_f32)
           + jnp.dot(R, Mr, preferred_element_type=_f32)
           + ds * bs + dr * br + bn1_ref[...])
    nh = jnp.maximum(pre, 0.0)
    out_ref[...] = jnp.dot(nh, wn2_ref[...], preferred_element_type=_f32) + bn2_ref[...]


def _node_mlp(h_nodes, S2, R2, D2, Wn1_n, We2, Wn1_s, Wn1_r, be2, bn1, Wn2, bn2):
    blk = 2048
    grid = (N // blk,)
    wspec = pl.BlockSpec((LAT, LAT), lambda i: (0, 0))
    bspec = pl.BlockSpec((1, LAT), lambda i: (0, 0))
    return pl.pallas_call(
        _node_mlp_body,
        grid=grid,
        in_specs=[
            pl.BlockSpec((blk, LAT), lambda i: (i, 0)),
            pl.BlockSpec((NC, blk, LAT), lambda i: (0, i, 0)),
            pl.BlockSpec((NC, blk, LAT), lambda i: (0, i, 0)),
            pl.BlockSpec((NC, 2, blk, L), lambda i: (0, 0, i, 0)),
            wspec, wspec, wspec, wspec, bspec, bspec, wspec, bspec,
        ],
        out_specs=pl.BlockSpec((blk, LAT), lambda i: (i, 0)),
        out_shape=jax.ShapeDtypeStruct((N, LAT), _f32),
    )(h_nodes, S2, R2, D2, Wn1_n, We2, Wn1_s, Wn1_r, be2, bn1, Wn2, bn2)


def _enc1_body(x_ref, w_ref, b_ref, out_ref):
    k = pl.program_id(0)

    @pl.when(k == 0)
    def _():
        out_ref[...] = jnp.zeros_like(out_ref)

    out_ref[...] += jnp.dot(x_ref[...], w_ref[...], preferred_element_type=_f32)

    @pl.when(k == pl.num_programs(0) - 1)
    def _():
        out_ref[...] = jnp.maximum(out_ref[...] + b_ref[...], 0.0)


def _enc1(X, Wenc1, benc1):
    kblk = 4096
    grid = (X.shape[1] // kblk,)
    return pl.pallas_call(
        _enc1_body,
        grid=grid,
        in_specs=[
            pl.BlockSpec((64, kblk), lambda k: (0, k)),
            pl.BlockSpec((kblk, 256), lambda k: (k, 0)),
            pl.BlockSpec((1, 256), lambda k: (0, 0)),
        ],
        out_specs=pl.BlockSpec((64, 256), lambda k: (0, 0)),
        out_shape=jax.ShapeDtypeStruct((64, 256), _f32),
    )(X, Wenc1, benc1)


def _enc2_body(x_ref, w_ref, b_ref, out_ref):
    out_ref[...] = jnp.maximum(
        jnp.dot(x_ref[...], w_ref[...], preferred_element_type=_f32) + b_ref[...], 0.0)


def _enc2(X, Wenc2, benc2):
    return pl.pallas_call(
        _enc2_body,
        out_shape=jax.ShapeDtypeStruct((64, 128), _f32),
    )(X, Wenc2, benc2)


def _sc_body(sidx_hbm, ridx_hbm, zeros_hbm, ep_hbm, p_hbm, q_hbm,
             s_out, r_out, d_out,
             s_idx, r_idx, fsb, frb, pb, qb, epb, ehb, shS, shR,
             sem_g0, sem_g1, sem_s0, sem_s1, sem_d):
    cid = lax.axis_index("c")
    sid = lax.axis_index("s")
    wid = sid * NC + cid
    row0 = wid * (EPT // 128)
    sem_g = (sem_g0, sem_g1)
    sem_s = (sem_s0, sem_s1)

    pltpu.sync_copy(sidx_hbm.at[pl.ds(wid * EPT, EPT)], s_idx)
    pltpu.sync_copy(ridx_hbm.at[pl.ds(wid * EPT, EPT)], r_idx)

    def _clear_tables():
        pltpu.async_copy(zeros_hbm, shS.at[pl.ds(sid * NODE_SL, NODE_SL)], sem_d)
        pltpu.async_copy(zeros_hbm, shR.at[pl.ds(sid * NODE_SL, NODE_SL)], sem_d)
        pltpu.make_async_copy(zeros_hbm, shS.at[pl.ds(sid * NODE_SL, NODE_SL)],
                              sem_d).wait()
        pltpu.make_async_copy(zeros_hbm, shR.at[pl.ds(sid * NODE_SL, NODE_SL)],
                              sem_d).wait()

    _clear_tables()
    plsc.subcore_barrier()

    def _drain_group(g):
        nrow = sid * NODE_SL
        pltpu.async_copy(shS.at[pl.ds(nrow, NODE_SL)],
                         s_out.at[cid, pl.ds(nrow, NODE_SL), pl.ds(g * L, L)],
                         sem_d)
        pltpu.async_copy(shR.at[pl.ds(nrow, NODE_SL)],
                         r_out.at[cid, pl.ds(nrow, NODE_SL), pl.ds(g * L, L)],
                         sem_d)
        pltpu.make_async_copy(
            shS.at[pl.ds(nrow, NODE_SL)],
            s_out.at[cid, pl.ds(nrow, NODE_SL), pl.ds(g * L, L)], sem_d).wait()
        pltpu.make_async_copy(
            shR.at[pl.ds(nrow, NODE_SL)],
            r_out.at[cid, pl.ds(nrow, NODE_SL), pl.ds(g * L, L)], sem_d).wait()
        _clear_tables()
        plsc.subcore_barrier()

    for g in range(G):
        def _ep_src(ch):
            e0 = wid * EPT + ch * CH
            return ep_hbm.at[pl.ds(e0, CH), pl.ds(g * L, L)]

        def _issue(ch, par):
            def _fidx(c, _):
                sl = pl.ds(c * 16, 16)
                src_sl = pl.ds(ch * CH + c * 16, 16)
                fsb[par, sl] = s_idx[src_sl] * G + g
                frb[par, sl] = r_idx[src_sl] * G + g
                return _
            lax.fori_loop(0, CH // 16, _fidx, None)
            pltpu.async_copy(_ep_src(ch), epb.at[par], sem_g[par])
            pltpu.async_copy(p_hbm.at[fsb.at[par]], pb.at[par], sem_g[par])
            pltpu.async_copy(q_hbm.at[frb.at[par]], qb.at[par], sem_g[par])

        def _wait_gathers(ch, par):
            pltpu.make_async_copy(_ep_src(ch), epb.at[par], sem_g[par]).wait()
            pltpu.make_async_copy(p_hbm.at[fsb.at[par]], pb.at[par],
                                  sem_g[par]).wait()
            pltpu.make_async_copy(q_hbm.at[frb.at[par]], qb.at[par],
                                  sem_g[par]).wait()

        def _wait_scatters(ch, par):
            pltpu.make_async_copy(ehb.at[par],
                                  shS.at[s_idx.at[pl.ds(ch * CH, CH)]],
                                  sem_s[par]).wait()
            pltpu.make_async_copy(ehb.at[par],
                                  shR.at[r_idx.at[pl.ds(ch * CH, CH)]],
                                  sem_s[par]).wait()

        def _process(ch, par):
            _wait_gathers(ch, par)

            @pl.when(ch >= 2)
            def _():
                _wait_scatters(ch - 2, par)

            def _rows(i, _):
                for u in range(4):
                    r = i * 4 + u
                    ehb[par, r] = jnp.maximum(pb[par, r] + qb[par, r]
                                              + epb[par, r], 0.0)
                return _
            lax.fori_loop(0, CH // 4, _rows, None)

            pltpu.async_copy(ehb.at[par], shS.at[s_idx.at[pl.ds(ch * CH, CH)]],
                             sem_s[par], add=True)
            pltpu.async_copy(ehb.at[par], shR.at[r_idx.at[pl.ds(ch * CH, CH)]],
                             sem_s[par], add=True)

        _issue(0, 0)
        if g > 0:
            _drain_group(g - 1)

        def _pipe(i, _):
            c0 = i * 2
            _issue(c0 + 1, 1)
            _process(c0, 0)

            @pl.when(c0 + 2 < NCHUNK)
            def _():
                _issue(c0 + 2, 0)
            _process(c0 + 1, 1)
            return _
        lax.fori_loop(0, NCHUNK // 2, _pipe, None)
        _wait_scatters(NCHUNK - 2, 0)
        _wait_scatters(NCHUNK - 1, 1)
        plsc.subcore_barrier()

    _drain_group(G - 1)

    one0 = jnp.where(lax.iota(jnp.int32, L) == 0, 1.0, 0.0).astype(_f32)

    def _preset(i, _):
        pb[0, i] = one0
        return _
    lax.fori_loop(0, CH, _preset, None)

    def _dchunk(ch, _):
        pltpu.async_copy(pb.at[0], shS.at[s_idx.at[pl.ds(ch * CH, CH)]],
                         sem_s0, add=True)
        pltpu.async_copy(pb.at[0], shR.at[r_idx.at[pl.ds(ch * CH, CH)]],
                         sem_s0, add=True)
        return _
    lax.fori_loop(0, NCHUNK, _dchunk, None)

    def _ddrain(ch, _):
        pltpu.make_async_copy(pb.at[0], shS.at[s_idx.at[pl.ds(ch * CH, CH)]],
                              sem_s0).wait()
        pltpu.make_async_copy(pb.at[0], shR.at[r_idx.at[pl.ds(ch * CH, CH)]],
                              sem_s0).wait()
        return _
    lax.fori_loop(0, NCHUNK, _ddrain, None)
    plsc.subcore_barrier()
    nrow = sid * NODE_SL
    pltpu.sync_copy(shS.at[pl.ds(nrow, NODE_SL)], d_out.at[cid, 0, pl.ds(nrow, NODE_SL)])
    pltpu.sync_copy(shR.at[pl.ds(nrow, NODE_SL)], d_out.at[cid, 1, pl.ds(nrow, NODE_SL)])


@functools.cache
def _make_sc_kernel():
    return functools.partial(
        pl.kernel,
        out_type=(
            jax.ShapeDtypeStruct((NC, N, LAT), _f32),
            jax.ShapeDtypeStruct((NC, N, LAT), _f32),
            jax.ShapeDtypeStruct((NC, 2, N, L), _f32),
        ),
        mesh=plsc.VectorSubcoreMesh(core_axis_name="c", subcore_axis_name="s"),
        scratch_types=(
            pltpu.VMEM((EPT,), jnp.int32),
            pltpu.VMEM((EPT,), jnp.int32),
            pltpu.VMEM((2, CH), jnp.int32),
            pltpu.VMEM((2, CH), jnp.int32),
            pltpu.VMEM((2, CH, L), _f32),
            pltpu.VMEM((2, CH, L), _f32),
            pltpu.VMEM((2, CH, L), _f32),
            pltpu.VMEM((2, CH, L), _f32),
            pltpu.VMEM_SHARED((N, L), _f32),
            pltpu.VMEM_SHARED((N, L), _f32),
            pltpu.SemaphoreType.DMA,
            pltpu.SemaphoreType.DMA,
            pltpu.SemaphoreType.DMA,
            pltpu.SemaphoreType.DMA,
            pltpu.SemaphoreType.DMA,
        ),
        compiler_params=pltpu.CompilerParams(use_tc_tiling_on_sc=False),
    )(_sc_body)


def _sc_edge_pass(sidx, ridx, zeros_in, ep, p_flat, q_flat):
    return _make_sc_kernel()(sidx, ridx, zeros_in, ep, p_flat, q_flat)


def kernel(nodes, edges, senders, receivers, n_node, n_edge,
           W_embed_n, b_embed_n, W_embed_e, b_embed_e,
           We1, be1, We2, be2, Wn1, bn1, Wn2, bn2,
           Wenc1, benc1, Wenc2, benc2):
    We1_e, We1_s, We1_r = We1[0:128], We1[128:256], We1[256:384]
    Wn1_n, Wn1_s, Wn1_r = Wn1[0:128], Wn1[128:256], Wn1[256:384]

    h_nodes, P, Q = _node_proj(nodes, W_embed_n, b_embed_n.reshape(1, LAT),
                               We1_s, We1_r)
    Ep = _edge_proj(edges, W_embed_e, b_embed_e.reshape(1, LAT), We1_e,
                    be1.reshape(1, LAT))

    zeros_in = jnp.zeros((NODE_SL, L), _f32)

    S2, R2, D2 = _sc_edge_pass(senders, receivers, zeros_in, Ep,
                               P.reshape(N * G, L), Q.reshape(N * G, L))

    new_nodes = _node_mlp(h_nodes, S2, R2, D2, Wn1_n, We2, Wn1_s, Wn1_r,
                          be2.reshape(1, LAT), bn1.reshape(1, LAT), Wn2,
                          bn2.reshape(1, LAT))

    X = _enc1(new_nodes.reshape(64, 512 * LAT), Wenc1, benc1.reshape(1, 256))
    Y = _enc2(X, Wenc2, benc2.reshape(1, 128))
    return Y.reshape(8, 8, 128)
